# Initial kernel scaffold; baseline (speedup 1.0000x reference)
import functools

import jax
import jax.numpy as jnp
from jax import lax
from jax.experimental import pallas as pl
from jax.experimental.pallas import tpu as pltpu
from jax.experimental.pallas import tpu_sc as plsc

N_NODES = 10000
D = 128
CHUNK = 128
NC, NS = 2, 16
NW = NC * NS
N_SP = 10016
ROWS_PER_TILE = N_NODES // NS
ZROWS = N_SP // NS
MM_BLOCK = 1000


def _mm_body(x_ref, w_ref, o_ref):
    o_ref[...] = jnp.dot(x_ref[...], w_ref[...], preferred_element_type=jnp.float32)


def _summ_body(a_ref, b_ref, w_ref, o_ref):
    o_ref[...] = jnp.dot(a_ref[...] + b_ref[...], w_ref[...],
                         preferred_element_type=jnp.float32)


def _add_body(a_ref, b_ref, o_ref):
    o_ref[...] = a_ref[...] + b_ref[...]


def _tc_matmul(x, w):
    grid = (N_NODES // MM_BLOCK,)
    return pl.pallas_call(
        _mm_body,
        grid=grid,
        in_specs=[
            pl.BlockSpec((MM_BLOCK, D), lambda i: (i, 0)),
            pl.BlockSpec((D, D), lambda i: (0, 0)),
        ],
        out_specs=pl.BlockSpec((MM_BLOCK, D), lambda i: (i, 0)),
        out_shape=jax.ShapeDtypeStruct((N_NODES, D), jnp.float32),
    )(x, w)


def _tc_sum_matmul(p, w):
    grid = (N_NODES // MM_BLOCK,)
    return pl.pallas_call(
        _summ_body,
        grid=grid,
        in_specs=[
            pl.BlockSpec((MM_BLOCK, D), lambda i: (i, 0)),
            pl.BlockSpec((MM_BLOCK, D), lambda i: (i, 0)),
            pl.BlockSpec((D, D), lambda i: (0, 0)),
        ],
        out_specs=pl.BlockSpec((MM_BLOCK, D), lambda i: (i, 0)),
        out_shape=jax.ShapeDtypeStruct((N_NODES, D), jnp.float32),
    )(p[0], p[1], w)


def _tc_sum(p):
    grid = (N_NODES // MM_BLOCK,)
    return pl.pallas_call(
        _add_body,
        grid=grid,
        in_specs=[
            pl.BlockSpec((MM_BLOCK, D), lambda i: (i, 0)),
            pl.BlockSpec((MM_BLOCK, D), lambda i: (i, 0)),
        ],
        out_specs=pl.BlockSpec((MM_BLOCK, D), lambda i: (i, 0)),
        out_shape=jax.ShapeDtypeStruct((N_NODES, D), jnp.float32),
    )(p[0], p[1])


def _make_sc_segsum(n_chunks):
    mesh = plsc.VectorSubcoreMesh(core_axis_name="c", subcore_axis_name="s")

    @functools.partial(
        pl.kernel,
        mesh=mesh,
        out_type=jax.ShapeDtypeStruct((NC, N_NODES, D), jnp.float32),
        scratch_types=[
            pltpu.VMEM((n_chunks, CHUNK), jnp.int32),
            pltpu.VMEM((n_chunks, CHUNK), jnp.int32),
            pltpu.VMEM((CHUNK, D), jnp.float32),
            pltpu.VMEM((CHUNK, D), jnp.float32),
            pltpu.VMEM_SHARED((N_SP, D), jnp.float32),
            pltpu.SemaphoreType.DMA,
        ],
    )
    def segsum(h_hbm, src_hbm, dst_hbm, out_hbm, src_v, dst_v, rows_v, zero_v,
               acc_sh, sem):
        c = lax.axis_index("c")
        s = lax.axis_index("s")
        wid = s * NC + c

        pltpu.sync_copy(src_hbm.at[wid], src_v)
        pltpu.sync_copy(dst_hbm.at[wid], dst_v)

        z = jnp.zeros((16,), jnp.float32)

        def _zero_row(i, _):
            for k in range(D // 16):
                zero_v[i, pl.ds(k * 16, 16)] = z
            return 0

        lax.fori_loop(0, CHUNK, _zero_row, 0)
        zbase = s * ZROWS
        nfull = ZROWS // CHUNK
        for j in range(nfull):
            pltpu.sync_copy(zero_v, acc_sh.at[pl.ds(zbase + j * CHUNK, CHUNK)])
        rem = ZROWS - nfull * CHUNK
        if rem:
            pltpu.sync_copy(zero_v.at[pl.ds(0, rem)],
                            acc_sh.at[pl.ds(zbase + nfull * CHUNK, rem)])
        plsc.subcore_barrier()

        def _step(j, _):
            pltpu.async_copy(h_hbm.at[src_v.at[j]], rows_v, sem).wait()
            pltpu.sync_copy(rows_v, acc_sh.at[dst_v.at[j]], add=True)
            return 0

        lax.fori_loop(0, n_chunks, _step, 0)
        plsc.subcore_barrier()

        pltpu.sync_copy(acc_sh.at[pl.ds(s * ROWS_PER_TILE, ROWS_PER_TILE)],
                        out_hbm.at[c, pl.ds(s * ROWS_PER_TILE, ROWS_PER_TILE)])

    return segsum


def kernel(x, edge_index, W1, W2, W3):
    src = jnp.asarray(edge_index[0], jnp.int32)
    dst = jnp.asarray(edge_index[1], jnp.int32)
    n_edges = src.shape[0]
    per_xfer = NW * CHUNK
    n_chunks = -(-n_edges // per_xfer)
    e_pad = n_chunks * per_xfer
    pad = e_pad - n_edges
    if pad:
        src = jnp.concatenate([src, jnp.zeros((pad,), jnp.int32)])
        dst = jnp.concatenate([dst, jnp.full((pad,), N_NODES, jnp.int32)])
    src3 = src.reshape(NW, n_chunks, CHUNK)
    dst3 = dst.reshape(NW, n_chunks, CHUNK)

    segsum = _make_sc_segsum(n_chunks)

    h = _tc_matmul(x, W1)
    p = segsum(h, src3, dst3)
    h = _tc_sum_matmul(p, W2)
    p = segsum(h, src3, dst3)
    h = _tc_sum_matmul(p, W3)
    p = segsum(h, src3, dst3)
    return _tc_sum(p)


# speedup vs baseline: 4.6378x; 4.6378x over previous
import functools

import jax
import jax.numpy as jnp
from jax import lax
from jax.experimental import pallas as pl
from jax.experimental.pallas import tpu as pltpu
from jax.experimental.pallas import tpu_sc as plsc

N_NODES = 10000
D = 128
CHUNK = 128
NC, NS = 2, 16
NW = NC * NS
N_SP = 10112
ROWS_PER_TILE = N_SP // NS
ZROWS = ROWS_PER_TILE
MM_BLOCK = 1000


def _mm_body(x_ref, w_ref, o_ref):
    o_ref[...] = jnp.dot(x_ref[...], w_ref[...], preferred_element_type=jnp.float32)


def _summ_body(a_ref, b_ref, w_ref, o_ref):
    o_ref[...] = jnp.dot(a_ref[...] + b_ref[...], w_ref[...],
                         preferred_element_type=jnp.float32)


def _add_body(a_ref, b_ref, o_ref):
    o_ref[...] = a_ref[...] + b_ref[...]


def _tc_matmul(x, w):
    grid = (N_NODES // MM_BLOCK,)
    return pl.pallas_call(
        _mm_body,
        grid=grid,
        in_specs=[
            pl.BlockSpec((MM_BLOCK, D), lambda i: (i, 0)),
            pl.BlockSpec((D, D), lambda i: (0, 0)),
        ],
        out_specs=pl.BlockSpec((MM_BLOCK, D), lambda i: (i, 0)),
        out_shape=jax.ShapeDtypeStruct((N_NODES, D), jnp.float32),
    )(x, w)


def _tc_sum_matmul(p, w):
    grid = (N_NODES // MM_BLOCK,)
    return pl.pallas_call(
        _summ_body,
        grid=grid,
        in_specs=[
            pl.BlockSpec((MM_BLOCK, D), lambda i: (i, 0)),
            pl.BlockSpec((MM_BLOCK, D), lambda i: (i, 0)),
            pl.BlockSpec((D, D), lambda i: (0, 0)),
        ],
        out_specs=pl.BlockSpec((MM_BLOCK, D), lambda i: (i, 0)),
        out_shape=jax.ShapeDtypeStruct((N_NODES, D), jnp.float32),
    )(p[0], p[1], w)


def _tc_sum(p):
    grid = (N_NODES // MM_BLOCK,)
    return pl.pallas_call(
        _add_body,
        grid=grid,
        in_specs=[
            pl.BlockSpec((MM_BLOCK, D), lambda i: (i, 0)),
            pl.BlockSpec((MM_BLOCK, D), lambda i: (i, 0)),
        ],
        out_specs=pl.BlockSpec((MM_BLOCK, D), lambda i: (i, 0)),
        out_shape=jax.ShapeDtypeStruct((N_NODES, D), jnp.float32),
    )(p[0], p[1])


def _make_sc_segsum(n_chunks):
    mesh = plsc.VectorSubcoreMesh(core_axis_name="c", subcore_axis_name="s")

    @functools.partial(
        pl.kernel,
        mesh=mesh,
        out_type=jax.ShapeDtypeStruct((NC, N_SP, D), jnp.float32),
        scratch_types=[
            pltpu.VMEM((n_chunks, CHUNK), jnp.int32),
            pltpu.VMEM((n_chunks, CHUNK), jnp.int32),
            pltpu.VMEM((CHUNK, D), jnp.float32),
            pltpu.VMEM_SHARED((N_SP, D), jnp.float32),
            pltpu.SemaphoreType.DMA,
        ],
    )
    def segsum(h_hbm, src_hbm, dst_hbm, out_hbm, src_v, dst_v, rows_v,
               acc_sh, sem):
        c = lax.axis_index("c")
        s = lax.axis_index("s")
        wid = s * NC + c

        pltpu.sync_copy(src_hbm.at[wid], src_v)
        pltpu.sync_copy(dst_hbm.at[wid], dst_v)

        z = jnp.zeros((16,), jnp.float32)

        def _zero_row(i, _):
            for k in range(D // 16):
                rows_v[i, pl.ds(k * 16, 16)] = z
            return 0

        lax.fori_loop(0, CHUNK, _zero_row, 0)
        zbase = s * ZROWS
        nfull = ZROWS // CHUNK
        for j in range(nfull):
            pltpu.sync_copy(rows_v, acc_sh.at[pl.ds(zbase + j * CHUNK, CHUNK)])
        rem = ZROWS - nfull * CHUNK
        if rem:
            pltpu.sync_copy(rows_v.at[pl.ds(0, rem)],
                            acc_sh.at[pl.ds(zbase + nfull * CHUNK, rem)])
        plsc.subcore_barrier()

        def _step(j, _):
            pltpu.async_copy(h_hbm.at[src_v.at[j]], rows_v, sem).wait()
            pltpu.sync_copy(rows_v, acc_sh.at[dst_v.at[j]], add=True)
            return 0

        lax.fori_loop(0, n_chunks, _step, 0)
        plsc.subcore_barrier()

        pltpu.sync_copy(acc_sh.at[pl.ds(s * ROWS_PER_TILE, ROWS_PER_TILE)],
                        out_hbm.at[c, pl.ds(s * ROWS_PER_TILE, ROWS_PER_TILE)])

    return segsum


def kernel(x, edge_index, W1, W2, W3):
    src = jnp.asarray(edge_index[0], jnp.int32)
    dst = jnp.asarray(edge_index[1], jnp.int32)
    n_edges = src.shape[0]
    per_xfer = NW * CHUNK
    n_chunks = -(-n_edges // per_xfer)
    e_pad = n_chunks * per_xfer
    pad = e_pad - n_edges
    if pad:
        src = jnp.concatenate([src, jnp.zeros((pad,), jnp.int32)])
        dst = jnp.concatenate([dst, jnp.full((pad,), N_NODES, jnp.int32)])
    src3 = src.reshape(NW, n_chunks, CHUNK)
    dst3 = dst.reshape(NW, n_chunks, CHUNK)

    segsum = _make_sc_segsum(n_chunks)

    h = _tc_matmul(x, W1)
    p = segsum(h, src3, dst3)
    h = _tc_sum_matmul(p, W2)
    p = segsum(h, src3, dst3)
    h = _tc_sum_matmul(p, W3)
    p = segsum(h, src3, dst3)
    return _tc_sum(p)
